# baseline (speedup 1.0000x reference)
import jax
import jax.numpy as jnp
from jax import lax
from jax.experimental import pallas as pl
from jax.experimental.pallas import tpu as pltpu
from jax.experimental.pallas import tpu_sc as plsc

B = 16384
Z = 64
L = 16
NC = 2
NS = 16
NW = NC * NS
BPW = B // NW
IDX_CHUNK = 128
N_CHUNKS = BPW // IDX_CHUNK


def _sc_body(i_hbm, sl_hbm, eps_hbm, mean_hbm, z_hbm, kl_hbm,
             idx_v, rows_v, eps_v, sl_v, kl_v, sem):
    wid = lax.axis_index("s") * NC + lax.axis_index("c")

    pltpu.sync_copy(i_hbm.at[wid], idx_v)
    copies = []
    for j in range(N_CHUNKS):
        copies.append(pltpu.async_copy(
            mean_hbm.at[idx_v.at[j]],
            rows_v.at[pl.ds(j * IDX_CHUNK, IDX_CHUNK)],
            sem,
        ))
    pltpu.sync_copy(eps_hbm.at[wid], eps_v)
    pltpu.sync_copy(sl_hbm, sl_v)

    sigmas = []
    c_acc = jnp.zeros((L,), jnp.float32)
    for j in range(Z // L):
        slj = sl_v[pl.ds(j * L, L)]
        sg = jnp.exp(slj)
        sigmas.append(sg)
        c_acc = c_acc + 0.5 * (sg * sg - 2.0 * slj - 1.0)
    c_const = c_acc[0]
    for k in range(1, L):
        c_const = c_const + c_acc[k]

    for c in copies:
        c.wait()

    lane = lax.iota(jnp.int32, L)

    def body(g, carry):
        base_r = g * L
        row_idx = base_r + lane
        klacc = jnp.zeros((L,), jnp.float32)
        for z in range(Z):
            col = plsc.load_gather(
                rows_v, [row_idx, jnp.full((L,), z, jnp.int32)])
            klacc = klacc + col * col
        kl_v[pl.ds(base_r, L)] = c_const + 0.5 * klacc
        for rr in range(L):
            r = base_r + rr
            for j in range(Z // L):
                mu = rows_v[r, pl.ds(j * L, L)]
                e = eps_v[r, pl.ds(j * L, L)]
                rows_v[r, pl.ds(j * L, L)] = mu + e * sigmas[j]
        return carry

    lax.fori_loop(0, BPW // L, body, 0)

    pltpu.sync_copy(rows_v, z_hbm.at[wid])
    pltpu.sync_copy(kl_v, kl_hbm.at[wid])


def kernel(i, mean, std_logits, eps):
    idx = i.reshape(NW, N_CHUNKS, IDX_CHUNK)
    sl = std_logits.reshape(Z)
    eps3 = eps.reshape(NW, BPW, Z)
    mesh = plsc.VectorSubcoreMesh(core_axis_name="c", subcore_axis_name="s")
    f = pl.kernel(
        _sc_body,
        mesh=mesh,
        out_type=[
            jax.ShapeDtypeStruct((NW, BPW, Z), jnp.float32),
            jax.ShapeDtypeStruct((NW, BPW), jnp.float32),
        ],
        scratch_types=[
            pltpu.VMEM((N_CHUNKS, IDX_CHUNK), jnp.int32),
            pltpu.VMEM((BPW, Z), jnp.float32),
            pltpu.VMEM((BPW, Z), jnp.float32),
            pltpu.VMEM((Z,), jnp.float32),
            pltpu.VMEM((BPW,), jnp.float32),
            pltpu.SemaphoreType.DMA,
        ],
        compiler_params=pltpu.CompilerParams(
            needs_layout_passes=False, use_tc_tiling_on_sc=False),
    )
    z, kl = f(idx, sl, eps3, mean)
    return z.reshape(B, Z), kl.reshape(B, 1)


# speedup vs baseline: 1.6234x; 1.6234x over previous
import jax
import jax.numpy as jnp
from jax import lax
from jax.experimental import pallas as pl
from jax.experimental.pallas import tpu as pltpu
from jax.experimental.pallas import tpu_sc as plsc

B = 16384
Z = 64
L = 16
NC = 2
NS = 16
NW = NC * NS
BPW = B // NW
HQ = BPW // 2
CH = 32
N_CH = HQ // CH


def _sc_body(i_hbm, sl_hbm, eps_hbm, mean_hbm, z_hbm, kl_hbm,
             idx_v, rows_v, eps_v, sl_v, kl_v, sem):
    wid = lax.axis_index("s") * NC + lax.axis_index("c")

    pltpu.sync_copy(i_hbm.at[wid], idx_v)
    pltpu.sync_copy(sl_hbm, sl_v)

    sigmas = []
    c_acc = jnp.zeros((L,), jnp.float32)
    for j in range(Z // L):
        slj = sl_v[pl.ds(j * L, L)]
        sg = jnp.exp(slj)
        sigmas.append(sg)
        c_acc = c_acc + 0.5 * (sg * sg - 2.0 * slj - 1.0)
    c_const = c_acc[0]
    for k in range(1, L):
        c_const = c_const + c_acc[k]

    lane = lax.iota(jnp.int32, L)

    for h in range(2):
        def issue_chunk(ch, carry):
            for k in range(CH // L):
                v = idx_v[h * N_CH + ch, pl.ds(k * L, L)]
                for t in range(L):
                    pltpu.async_copy(
                        mean_hbm.at[v[t]],
                        rows_v.at[ch * CH + k * L + t], sem)
            return carry

        lax.fori_loop(0, N_CH, issue_chunk, 0)

        pltpu.sync_copy(eps_hbm.at[wid, pl.ds(h * HQ, HQ)], eps_v)
        pltpu.make_async_copy(
            eps_hbm.at[wid, pl.ds(h * HQ, HQ)], rows_v, sem).wait()

        def body(g, carry):
            base_r = g * L
            row_idx = base_r + lane
            klacc = jnp.zeros((L,), jnp.float32)
            for z in range(Z):
                col = plsc.load_gather(
                    rows_v, [row_idx, jnp.full((L,), z, jnp.int32)])
                klacc = klacc + col * col
            kl_v[pl.ds(h * HQ + base_r, L)] = c_const + 0.5 * klacc
            for rr in range(L):
                r = base_r + rr
                for j in range(Z // L):
                    mu = rows_v[r, pl.ds(j * L, L)]
                    e = eps_v[r, pl.ds(j * L, L)]
                    rows_v[r, pl.ds(j * L, L)] = mu + e * sigmas[j]
            return carry

        lax.fori_loop(0, HQ // L, body, 0)

        pltpu.sync_copy(rows_v, z_hbm.at[wid, pl.ds(h * HQ, HQ)])

    pltpu.sync_copy(kl_v, kl_hbm.at[wid])


def kernel(i, mean, std_logits, eps):
    idx = i.reshape(NW, 2 * N_CH, CH)
    sl = std_logits.reshape(Z)
    eps3 = eps.reshape(NW, BPW, Z)
    mesh = plsc.VectorSubcoreMesh(core_axis_name="c", subcore_axis_name="s")
    f = pl.kernel(
        _sc_body,
        mesh=mesh,
        out_type=[
            jax.ShapeDtypeStruct((NW, BPW, Z), jnp.float32),
            jax.ShapeDtypeStruct((NW, BPW), jnp.float32),
        ],
        scratch_types=[
            pltpu.VMEM((2 * N_CH, CH), jnp.int32),
            pltpu.VMEM((HQ, Z), jnp.float32),
            pltpu.VMEM((HQ, Z), jnp.float32),
            pltpu.VMEM((Z,), jnp.float32),
            pltpu.VMEM((BPW,), jnp.float32),
            pltpu.SemaphoreType.DMA,
        ],
        compiler_params=pltpu.CompilerParams(needs_layout_passes=False),
    )
    z, kl = f(idx, sl, eps3, mean)
    return z.reshape(B, Z), kl.reshape(B, 1)
